# baseline (speedup 1.0000x reference)
import jax
import jax.numpy as jnp
from jax import lax
from jax.experimental import pallas as pl
from jax.experimental.pallas import tpu as pltpu
from jax.experimental.pallas import tpu_sc as plsc

_BATCH = 4096
_SEQ = 50
_D = 64
_DP = 128
_V = 1000000
_NC, _NS = 2, 16
_NW = _NC * _NS
_BB = _BATCH // _NW

_TBLK = 32768


def _tp_body(in_ref, out_ref):
    out_ref[:, 0:_D] = in_ref[...].T


def _emb_body(xt_hbm, table_hbm, out_hbm, idx_v, rows0, rows1, cb0, cb1,
              gsem0, gsem1, wsem0, wsem1):
    wid = lax.axis_index("s") * _NC + lax.axis_index("c")
    b0 = wid * _BB
    rows = (rows0, rows1)
    cbs = (cb0, cb1)
    gsems = (gsem0, gsem1)
    wsems = (wsem0, wsem1)

    pltpu.sync_copy(xt_hbm.at[:, pl.ds(b0, _BB)], idx_v)

    lanes = lax.iota(jnp.int32, 16)

    def extract(k):
        def grp(m, carry):
            ridx = m * 16 + lanes
            for f in range(_D):
                cidx = jnp.full((16,), f, jnp.int32)
                v = plsc.load_gather(rows[k], [ridx, cidx])
                cbs[k][f, pl.ds(m * 16, 16)] = v
            return carry

        lax.fori_loop(0, _BB // 16, grp, 0)

    pltpu.async_copy(table_hbm.at[idx_v.at[0]], rows[0], gsems[0])

    def body(i, carry):
        for half in range(2):
            s = i * 2 + half
            k = half
            nk = 1 - half

            @pl.when(s + 1 < _SEQ)
            def _start_next():
                pltpu.async_copy(
                    table_hbm.at[idx_v.at[s + 1]], rows[nk], gsems[nk])

            pltpu.make_async_copy(
                table_hbm.at[idx_v.at[s]], rows[k], gsems[k]).wait()
            extract(k)
            pltpu.async_copy(cbs[k], out_hbm.at[s, :, pl.ds(b0, _BB)],
                             wsems[k])
            pltpu.make_async_copy(
                cbs[k], out_hbm.at[s, :, pl.ds(b0, _BB)], wsems[k]).wait()
        return carry

    lax.fori_loop(0, _SEQ // 2, body, 0)


def kernel(x, table):
    grid = pl.cdiv(_V, _TBLK)
    t2 = pl.pallas_call(
        _tp_body,
        grid=(grid,),
        in_specs=[pl.BlockSpec((_D, _TBLK), lambda j: (0, j))],
        out_specs=pl.BlockSpec((_TBLK, _DP), lambda j: (j, 0)),
        out_shape=jax.ShapeDtypeStruct((_V, _DP), jnp.float32),
    )(table.T)
    mesh = plsc.VectorSubcoreMesh(core_axis_name="c", subcore_axis_name="s")
    out3 = pl.kernel(
        _emb_body,
        out_type=jax.ShapeDtypeStruct((_SEQ, _D, _BATCH), jnp.float32),
        mesh=mesh,
        scratch_types=[
            pltpu.VMEM((_SEQ, _BB), jnp.int32),
            pltpu.VMEM((_BB, _DP), jnp.float32),
            pltpu.VMEM((_BB, _DP), jnp.float32),
            pltpu.VMEM((_D, _BB), jnp.float32),
            pltpu.VMEM((_D, _BB), jnp.float32),
            pltpu.SemaphoreType.DMA,
            pltpu.SemaphoreType.DMA,
            pltpu.SemaphoreType.DMA,
            pltpu.SemaphoreType.DMA,
        ],
        compiler_params=pltpu.CompilerParams(
            use_tc_tiling_on_sc=True, needs_layout_passes=False),
    )(x.T, t2)
    return jnp.transpose(out3, (2, 0, 1))


# speedup vs baseline: 1.0291x; 1.0291x over previous
import jax
import jax.numpy as jnp
from jax import lax
from jax.experimental import pallas as pl
from jax.experimental.pallas import tpu as pltpu
from jax.experimental.pallas import tpu_sc as plsc

_BATCH = 4096
_SEQ = 50
_D = 64
_DP = 128
_V = 1000000
_NC, _NS = 2, 16
_NW = _NC * _NS
_BB = _BATCH // _NW

_TBLK = 32768


def _tp_body(in_ref, out_ref):
    out_ref[:, 0:_D] = in_ref[...].T


def _emb_body(xt_hbm, table_hbm, out_hbm, idx_v, rows0, rows1, cb0, cb1,
              gsem0, gsem1, wsem0, wsem1):
    wid = lax.axis_index("s") * _NC + lax.axis_index("c")
    b0 = wid * _BB
    rows = (rows0, rows1)
    cbs = (cb0, cb1)
    gsems = (gsem0, gsem1)
    wsems = (wsem0, wsem1)

    pltpu.sync_copy(xt_hbm.at[:, pl.ds(b0, _BB)], idx_v)

    lanes = lax.iota(jnp.int32, 16)

    def extract(k):
        def grp(m, carry):
            ridx = m * 16 + lanes
            for f in range(_D):
                cidx = jnp.full((16,), f, jnp.int32)
                v = plsc.load_gather(rows[k], [ridx, cidx])
                cbs[k][f, pl.ds(m * 16, 16)] = v
            return carry

        lax.fori_loop(0, _BB // 16, grp, 0)

    pltpu.async_copy(table_hbm.at[idx_v.at[0]], rows[0], gsems[0])

    def body(i, carry):
        for half in range(2):
            s = i * 2 + half
            k = half
            nk = 1 - half

            @pl.when(s + 1 < _SEQ)
            def _start_next():
                pltpu.async_copy(
                    table_hbm.at[idx_v.at[s + 1]], rows[nk], gsems[nk])

            pltpu.make_async_copy(
                table_hbm.at[idx_v.at[s]], rows[k], gsems[k]).wait()

            @pl.when(s >= 2)
            def _drain_prev_wb():
                pltpu.make_async_copy(
                    cbs[k], out_hbm.at[s, :, pl.ds(b0, _BB)],
                    wsems[k]).wait()

            extract(k)
            pltpu.async_copy(cbs[k], out_hbm.at[s, :, pl.ds(b0, _BB)],
                             wsems[k])
        return carry

    lax.fori_loop(0, _SEQ // 2, body, 0)
    for k in range(2):
        pltpu.make_async_copy(
            cbs[k], out_hbm.at[_SEQ - 2 + k, :, pl.ds(b0, _BB)],
            wsems[k]).wait()


def kernel(x, table):
    grid = pl.cdiv(_V, _TBLK)
    t2 = pl.pallas_call(
        _tp_body,
        grid=(grid,),
        in_specs=[pl.BlockSpec((_D, _TBLK), lambda j: (0, j))],
        out_specs=pl.BlockSpec((_TBLK, _DP), lambda j: (j, 0)),
        out_shape=jax.ShapeDtypeStruct((_V, _DP), jnp.float32),
    )(table.T)
    mesh = plsc.VectorSubcoreMesh(core_axis_name="c", subcore_axis_name="s")
    out3 = pl.kernel(
        _emb_body,
        out_type=jax.ShapeDtypeStruct((_SEQ, _D, _BATCH), jnp.float32),
        mesh=mesh,
        scratch_types=[
            pltpu.VMEM((_SEQ, _BB), jnp.int32),
            pltpu.VMEM((_BB, _DP), jnp.float32),
            pltpu.VMEM((_BB, _DP), jnp.float32),
            pltpu.VMEM((_D, _BB), jnp.float32),
            pltpu.VMEM((_D, _BB), jnp.float32),
            pltpu.SemaphoreType.DMA,
            pltpu.SemaphoreType.DMA,
            pltpu.SemaphoreType.DMA,
            pltpu.SemaphoreType.DMA,
        ],
        compiler_params=pltpu.CompilerParams(
            use_tc_tiling_on_sc=True, needs_layout_passes=False),
    )(x.T, t2)
    return jnp.transpose(out3, (2, 0, 1))
